# baseline (speedup 1.0000x reference)
import functools

import numpy as np
import jax
import jax.numpy as jnp
from jax.experimental import pallas as pl
from jax.experimental.pallas import tpu as pltpu


def _crack_pixels(B, H, W, n_cracks, seed=0):
    rng = np.random.default_rng(seed)
    bs, ys, xs = [], [], []
    for b in range(B):
        for _ in range(n_cracks):
            y0 = int(rng.integers(0, H)); x0 = int(rng.integers(0, W))
            y1 = int(rng.integers(0, H)); x1 = int(rng.integers(0, W))
            dx, dy = abs(x1 - x0), abs(y1 - y0)
            sx = 1 if x0 < x1 else -1
            sy = 1 if y0 < y1 else -1
            err = dx - dy
            cx, cy = x0, y0
            for _ in range(max(dx, dy) + 1):
                if 0 <= cy < H and 0 <= cx < W:
                    bs.append(b); ys.append(cy); xs.append(cx)
                e2 = 2 * err
                if e2 > -dy:
                    err -= dy; cx += sx
                if e2 < dx:
                    err += dx; cy += sy
    return (np.asarray(bs, dtype=np.int64),
            np.asarray(ys, dtype=np.int64),
            np.asarray(xs, dtype=np.int64))


@functools.lru_cache(maxsize=None)
def _crack_mask_np(B, H, W, n_cracks):
    bs, ys, xs = _crack_pixels(B, H, W, n_cracks)
    m = np.zeros((B, 1, H, W), dtype=np.bool_)
    m[bs, 0, ys, xs] = True
    return m


def _body(m_ref, x_ref, o_ref):
    x = x_ref[...]
    m = m_ref[...]
    o_ref[...] = jnp.where(m, jnp.float32(0.05),
                           jnp.clip(x, jnp.float32(0.0), jnp.float32(1.0)))


def kernel(x):
    B, C, H, W = x.shape
    mask = jnp.asarray(_crack_mask_np(B, H, W, 5))
    out = pl.pallas_call(
        _body,
        grid=(B // 4,),
        in_specs=[
            pl.BlockSpec((4, 1, H, W), lambda b: (b, 0, 0, 0)),
            pl.BlockSpec((4, C, H, W), lambda b: (b, 0, 0, 0)),
        ],
        out_specs=pl.BlockSpec((4, C, H, W), lambda b: (b, 0, 0, 0)),
        out_shape=jax.ShapeDtypeStruct((B, C, H, W), x.dtype),
    )(mask, x)
    return out


# speedup vs baseline: 1.0057x; 1.0057x over previous
import functools

import numpy as np
import jax
import jax.numpy as jnp
from jax.experimental import pallas as pl
from jax.experimental.pallas import tpu as pltpu


def _crack_pixels(B, H, W, n_cracks, seed=0):
    rng = np.random.default_rng(seed)
    bs, ys, xs = [], [], []
    for b in range(B):
        for _ in range(n_cracks):
            y0 = int(rng.integers(0, H)); x0 = int(rng.integers(0, W))
            y1 = int(rng.integers(0, H)); x1 = int(rng.integers(0, W))
            dx, dy = abs(x1 - x0), abs(y1 - y0)
            sx = 1 if x0 < x1 else -1
            sy = 1 if y0 < y1 else -1
            err = dx - dy
            cx, cy = x0, y0
            for _ in range(max(dx, dy) + 1):
                if 0 <= cy < H and 0 <= cx < W:
                    bs.append(b); ys.append(cy); xs.append(cx)
                e2 = 2 * err
                if e2 > -dy:
                    err -= dy; cx += sx
                if e2 < dx:
                    err += dx; cy += sy
    return (np.asarray(bs, dtype=np.int64),
            np.asarray(ys, dtype=np.int64),
            np.asarray(xs, dtype=np.int64))


@functools.lru_cache(maxsize=None)
def _crack_mask_np(B, H, W, n_cracks):
    bs, ys, xs = _crack_pixels(B, H, W, n_cracks)
    m = np.zeros((B, 1, H, W), dtype=np.bool_)
    m[bs, 0, ys, xs] = True
    return m


def _body(m_ref, x_ref, o_ref):
    x = x_ref[...]
    m = m_ref[...]
    o_ref[...] = jnp.where(m, jnp.float32(0.05),
                           jnp.clip(x, jnp.float32(0.0), jnp.float32(1.0)))


def kernel(x):
    B, C, H, W = x.shape
    mask = jnp.asarray(_crack_mask_np(B, H, W, 5))
    out = pl.pallas_call(
        _body,
        grid=(B // 2,),
        in_specs=[
            pl.BlockSpec((2, 1, H, W), lambda b: (b, 0, 0, 0)),
            pl.BlockSpec((2, C, H, W), lambda b: (b, 0, 0, 0)),
        ],
        out_specs=pl.BlockSpec((2, C, H, W), lambda b: (b, 0, 0, 0)),
        out_shape=jax.ShapeDtypeStruct((B, C, H, W), x.dtype),
    )(mask, x)
    return out
